# baseline (speedup 1.0000x reference)
import functools

import jax
import jax.numpy as jnp
from jax import lax
from jax.experimental import pallas as pl
from jax.experimental.pallas import tpu as pltpu
from jax.experimental.pallas import tpu_sc as plsc

N_NODES = 10000
N_PAD = 10240
E_EDGES = 320000
H = 128
HW = H // 2
OUT_DIM = 47

NW = 32
E_PER_W = E_EDGES // NW
GCHUNK = 200
NODES_PER_W = N_PAD // NW
DCHUNK = 2000

_sc_mesh = functools.partial(
    plsc.VectorSubcoreMesh, core_axis_name="c", subcore_axis_name="s")


def _wid():
    return lax.axis_index("s") * 2 + lax.axis_index("c")


@functools.partial(
    pl.kernel,
    mesh=_sc_mesh(),
    compiler_params=pltpu.CompilerParams(needs_layout_passes=False),
    out_type=jax.ShapeDtypeStruct((E_EDGES, 2 * HW), jnp.int32),
    scratch_types=[
        pltpu.VMEM((GCHUNK,), jnp.int32),
        pltpu.VMEM((GCHUNK,), jnp.int32),
        pltpu.VMEM((GCHUNK, 2 * HW), jnp.int32),
        pltpu.VMEM((GCHUNK, 2 * HW), jnp.int32),
        pltpu.SemaphoreType.DMA,
        pltpu.SemaphoreType.DMA,
    ],
)
def _sc_edge_gather(m1_hbm, m2_hbm, src_hbm, dst_hbm, out_hbm,
                    sidx, didx, b1, b2, sem1, sem2):
    base = _wid() * E_PER_W

    def chunk(j, carry):
        off = base + j * GCHUNK
        pltpu.sync_copy(src_hbm.at[pl.ds(off, GCHUNK)], sidx)
        pltpu.sync_copy(dst_hbm.at[pl.ds(off, GCHUNK)], didx)
        cp1 = pltpu.async_copy(m1_hbm.at[sidx], b1, sem1)
        cp2 = pltpu.async_copy(m2_hbm.at[didx], b2, sem2)
        cp1.wait()
        cp2.wait()

        def row(r, c2):
            for u in range(2 * HW // 16):
                sl = pl.ds(u * 16, 16)
                a = plsc.bitcast(b1[r, sl], jnp.bfloat16)
                b = plsc.bitcast(b2[r, sl], jnp.bfloat16)
                b1[r, sl] = plsc.bitcast(a + b, jnp.int32)
            return c2

        lax.fori_loop(0, GCHUNK, row, 0)
        pltpu.sync_copy(b1, out_hbm.at[pl.ds(off, GCHUNK)])
        return carry

    lax.fori_loop(0, E_PER_W // GCHUNK, chunk, 0)


RING = 4096
FBLK = 2048
EROUND = 158 * FBLK


@functools.partial(
    pl.kernel,
    mesh=_sc_mesh(),
    compiler_params=pltpu.CompilerParams(needs_layout_passes=False),
    out_type=(jax.ShapeDtypeStruct((NW * EROUND,), jnp.int32),
              jax.ShapeDtypeStruct((NW * 16,), jnp.int32)),
    scratch_types=[
        pltpu.VMEM((DCHUNK,), jnp.int32),
        pltpu.VMEM((RING,), jnp.int32),
        pltpu.VMEM((16,), jnp.int32),
    ],
)
def _sc_bin(dst_hbm, list_hbm, cnt_hbm, dbuf, ring, cbuf):
    wid = _wid()
    lo = wid * NODES_PER_W
    lane = lax.iota(jnp.int32, 16)
    lov = lax.broadcast_in_dim(lo, (16,), ())
    npw_vec = jnp.full((16,), NODES_PER_W, dtype=jnp.int32)
    dummy = jnp.full((16,), NODES_PER_W, dtype=jnp.int32)

    def init_ring(r, c):
        ring[pl.ds(r * 16, 16)] = dummy
        return c

    lax.fori_loop(0, RING // 16, init_ring, 0)

    def flush(state):
        cnt, flushed = state
        blk = flushed // FBLK

        def do(par):
            off = pl.multiple_of(wid * EROUND + flushed, FBLK)
            pltpu.sync_copy(ring.at[pl.ds(par * FBLK, FBLK)],
                            list_hbm.at[pl.ds(off, FBLK)])

        @pl.when(blk % 2 == 0)
        def _():
            do(0)

        @pl.when(blk % 2 == 1)
        def _():
            do(1)

        return cnt, flushed + FBLK

    def chunk(c, carry):
        cnt0, flushed, eid0 = carry
        pltpu.sync_copy(dst_hbm.at[pl.ds(c * DCHUNK, DCHUNK)], dbuf)

        def scan_group(g, st):
            cnt, eidv = st
            v = dbuf[pl.ds(g * 16, 16)]
            rel = v - lov
            m = jnp.bitwise_and(rel >= 0, rel < npw_vec)
            cs = jnp.cumsum(m.astype(jnp.int32))
            addr = jnp.bitwise_and(
                lax.broadcast_in_dim(cnt, (16,), ()) + cs - 1, RING - 1)
            packed = eidv * 512 + rel
            plsc.store_scatter(ring, [addr], packed, mask=m)
            return cnt + jnp.max(cs), eidv + 16

        cnt, eid0 = lax.fori_loop(0, DCHUNK // 16, scan_group, (cnt0, eid0))
        cnt, flushed = lax.cond(cnt - flushed >= FBLK, flush,
                                lambda s: s, (cnt, flushed))
        return cnt, flushed, eid0

    cnt, flushed, _ = lax.fori_loop(0, E_EDGES // DCHUNK, chunk,
                                    (jnp.int32(0), jnp.int32(0), lane))
    for _ in range(2):
        cnt, flushed = lax.cond(flushed < cnt, flush,
                                lambda s: s, (cnt, flushed))
    cbuf[...] = lax.broadcast_in_dim(cnt, (16,), ())
    pltpu.sync_copy(cbuf, cnt_hbm.at[pl.ds(pl.multiple_of(wid * 16, 16), 16)])


RB = 128
GPB = RB // 16


@functools.partial(
    pl.kernel,
    mesh=_sc_mesh(),
    compiler_params=pltpu.CompilerParams(needs_layout_passes=False),
    out_type=jax.ShapeDtypeStruct((N_PAD, 2 * HW), jnp.int32),
    scratch_types=[
        pltpu.VMEM((16,), jnp.int32),
        pltpu.VMEM((FBLK,), jnp.int32),
        pltpu.VMEM((FBLK,), jnp.int32),
        pltpu.VMEM((FBLK,), jnp.int32),
        pltpu.VMEM((RB, 2 * HW), jnp.int32),
        pltpu.VMEM((RB, 2 * HW), jnp.int32),
        pltpu.VMEM((NODES_PER_W + 1, 2 * HW), jnp.int32),
        pltpu.SemaphoreType.DMA,
        pltpu.SemaphoreType.DMA,
    ],
)
def _sc_segmax(p_hbm, list_hbm, cnt_hbm, red_hbm,
               cbuf, lbuf, idxb, relb, rb0, rb1, acc, sem0, sem1):
    wid = _wid()
    lo = wid * NODES_PER_W
    neg_inf = jnp.full((16,), 0xFF80FF80 - (1 << 32), dtype=jnp.int32)

    def init_row(r, c):
        for u in range(2 * HW // 16):
            acc[r, pl.ds(u * 16, 16)] = neg_inf
        return c

    lax.fori_loop(0, NODES_PER_W + 1, init_row, 0)

    pltpu.sync_copy(cnt_hbm.at[pl.ds(pl.multiple_of(wid * 16, 16), 16)], cbuf)
    cnt = cbuf[pl.ds(0, 16)][0]
    nlb = (cnt + FBLK - 1) // FBLK

    rbufs = (rb0, rb1)
    sems = (sem0, sem1)

    def fire(b, par):
        pltpu.async_copy(p_hbm.at[idxb.at[pl.ds(b * RB, RB)]],
                         rbufs[par], sems[par])

    def wait(par):
        pltpu.make_async_copy(p_hbm.at[idxb.at[pl.ds(0, RB)]],
                              rbufs[par], sems[par]).wait()

    def lblock(bidx, c0):
        loff = pl.multiple_of(wid * EROUND + bidx * FBLK, FBLK)
        pltpu.sync_copy(list_hbm.at[pl.ds(loff, FBLK)], lbuf)

        def decode(g, c):
            u = lbuf[pl.ds(g * 16, 16)]
            idxb[pl.ds(g * 16, 16)] = lax.shift_right_logical(u, 9)
            relb[pl.ds(g * 16, 16)] = jnp.bitwise_and(u, 511)
            return c

        lax.fori_loop(0, FBLK // 16, decode, 0)
        rem = jnp.minimum(cnt - bidx * FBLK, FBLK)
        ngroups = (rem + 15) // 16
        nb = (rem + RB - 1) // RB

        @pl.when(nb > 0)
        def _():
            fire(0, 0)

        @pl.when(nb > 1)
        def _():
            fire(1, 1)

        def bpair(k, c):
            for par in range(2):
                b = 2 * k + par

                @pl.when(b < nb)
                def _():
                    wait(par)
                    gend = jnp.minimum((b + 1) * GPB, ngroups)

                    def group(g, c2):
                        gl = g - b * GPB
                        relv = relb[pl.ds(g * 16, 16)]
                        for j in range(16):
                            rel = relv[j]
                            row = gl * 16 + j
                            for u in range(2 * HW // 16):
                                sl = pl.ds(u * 16, 16)
                                a = plsc.bitcast(acc[rel, sl], jnp.bfloat16)
                                r = plsc.bitcast(rbufs[par][row, sl],
                                                 jnp.bfloat16)
                                acc[rel, sl] = plsc.bitcast(
                                    jnp.maximum(a, r), jnp.int32)
                        return c2

                    lax.fori_loop(b * GPB, gend, group, 0)

                    @pl.when(b + 2 < nb)
                    def _():
                        fire(b + 2, par)
            return c

        lax.fori_loop(0, (nb + 1) // 2, bpair, 0)
        return c0

    lax.fori_loop(0, nlb, lblock, 0)
    pltpu.sync_copy(acc.at[pl.ds(0, NODES_PER_W)],
                    red_hbm.at[pl.ds(lo, NODES_PER_W)])


BN = 256
BE = 512


def _unpack_half(words_i32):
    u = lax.bitcast_convert_type(words_i32, jnp.uint32)
    even = lax.bitcast_convert_type(u << 16, jnp.float32)
    odd = lax.bitcast_convert_type(u & jnp.uint32(0xFFFF0000), jnp.float32)
    return jnp.concatenate([even, odd], axis=1)


def _pack_half(vals_f32):
    def rne(x):
        u = lax.bitcast_convert_type(x, jnp.uint32)
        return (u + jnp.uint32(0x7FFF) + ((u >> 16) & jnp.uint32(1))) >> 16

    ev = rne(vals_f32[:, :HW])
    od = rne(vals_f32[:, HW:])
    return lax.bitcast_convert_type(ev | (od << 16), jnp.int32)


def _tc_call(body, grid, in_specs, out_specs, out_shape):
    return pl.pallas_call(body, grid=grid, in_specs=in_specs,
                          out_specs=out_specs, out_shape=out_shape)


def _full(shape):
    return pl.BlockSpec(shape, lambda i: (0,) * len(shape))


def _rows(block, width):
    return pl.BlockSpec((block, width), lambda i: (i, 0))


def _tc_proj(x, w, b):
    def body(x_ref, w_ref, b_ref, o_ref):
        o_ref[...] = jnp.dot(x_ref[...], w_ref[...],
                             preferred_element_type=jnp.float32) + b_ref[...]
    return _tc_call(
        body, (N_PAD // BN,),
        [_rows(BN, H), _full((H, H)), _full((1, H))],
        _rows(BN, H), jax.ShapeDtypeStruct((N_PAD, H), jnp.float32))(x, w, b)


def _tc_pre(nf, hid, wm1, bm1, wm2, bm2, wo1, bo1):
    def body(nf_ref, hid_ref, wm1_ref, bm1_ref, wm2_ref, bm2_ref,
             wo1_ref, bo1_ref, m1_ref, m2_ref, h1_ref):
        z = jnp.concatenate([nf_ref[...], hid_ref[...]], axis=1)
        m1 = jnp.dot(z, wm1_ref[...],
                     preferred_element_type=jnp.float32) + bm1_ref[...]
        m2 = jnp.dot(z, wm2_ref[...],
                     preferred_element_type=jnp.float32) + bm2_ref[...]
        for s in range(2):
            sl = slice(s * H, (s + 1) * H)
            m1_ref[:, s * HW:(s + 1) * HW] = _pack_half(m1[:, sl])
            m2_ref[:, s * HW:(s + 1) * HW] = _pack_half(m2[:, sl])
        h1_ref[...] = jnp.dot(z, wo1_ref[...],
                              preferred_element_type=jnp.float32) + bo1_ref[...]

    shp_pk = jax.ShapeDtypeStruct((N_PAD, 2 * HW), jnp.int32)
    shp32 = jax.ShapeDtypeStruct((N_PAD, 2 * H), jnp.float32)
    return _tc_call(
        body, (N_PAD // BN,),
        [_rows(BN, H), _rows(BN, H),
         _full((2 * H, 2 * H)), _full((1, 2 * H)),
         _full((2 * H, 2 * H)), _full((1, 2 * H)),
         _full((2 * H, 2 * H)), _full((1, 2 * H))],
        (_rows(BN, 2 * HW), _rows(BN, 2 * HW), _rows(BN, 2 * H)),
        (shp_pk, shp_pk, shp32),
    )(nf, hid, wm1, bm1, wm2, bm2, wo1, bo1)


def _tc_edge_mlp(e, w1, b1, w2, b2):
    def body(e_ref, w1_ref, b1_ref, w2_ref, b2_ref, o_ref):
        for s in range(2):
            sl = slice(s * H, (s + 1) * H)
            wsl = slice(s * HW, (s + 1) * HW)
            msgs = jnp.maximum(_unpack_half(e_ref[:, wsl]), 0.0)
            t = jnp.maximum(
                jnp.dot(msgs, w1_ref[s], preferred_element_type=jnp.float32)
                + b1_ref[:, sl], 0.0)
            out = jnp.dot(t, w2_ref[s],
                          preferred_element_type=jnp.float32) + b2_ref[:, sl]
            o_ref[:, wsl] = _pack_half(out)

    return _tc_call(
        body, (E_EDGES // BE,),
        [_rows(BE, 2 * HW), _full((2, H, H)), _full((1, 2 * H)),
         _full((2, H, H)), _full((1, 2 * H))],
        _rows(BE, 2 * HW),
        jax.ShapeDtypeStruct((E_EDGES, 2 * HW), jnp.int32))(e, w1, b1, w2, b2)


def _tc_post(red, h1cat, wo2, bo2, ln_s, ln_b, wred, bred):
    def body(red_ref, h1_ref, wo2_ref, bo2_ref, lns_ref, lnb_ref,
             wred_ref, bred_ref, o_ref):
        outs = []
        for s in range(2):
            sl = slice(s * H, (s + 1) * H)
            r = _unpack_half(red_ref[:, s * HW:(s + 1) * HW])
            r = jnp.where(jnp.isfinite(r), r, 0.0)
            h2 = jnp.dot(r, wo2_ref[s],
                         preferred_element_type=jnp.float32) + bo2_ref[:, sl]
            ret = jnp.maximum(h1_ref[:, sl] + h2, 0.0)
            mu = jnp.mean(ret, axis=-1, keepdims=True)
            d = ret - mu
            var = jnp.mean(d * d, axis=-1, keepdims=True)
            ret = d / jnp.sqrt(var + 1e-5) * lns_ref[:, sl] + lnb_ref[:, sl]
            outs.append(ret)
        cat = jnp.concatenate(outs, axis=1)
        o_ref[...] = jnp.dot(cat, wred_ref[...],
                             preferred_element_type=jnp.float32) + bred_ref[...]

    return _tc_call(
        body, (N_PAD // BN,),
        [_rows(BN, 2 * HW), _rows(BN, 2 * H), _full((2, H, H)),
         _full((1, 2 * H)), _full((1, 2 * H)), _full((1, 2 * H)),
         _full((2 * H, H)), _full((1, H))],
        _rows(BN, H),
        jax.ShapeDtypeStruct((N_PAD, H), jnp.float32),
    )(red, h1cat, wo2, bo2, ln_s, ln_b, wred, bred)


def _tc_head(hid, wp1, bp1, wp2, bp2):
    def body(h_ref, wp1_ref, bp1_ref, wp2_ref, bp2_ref, o_ref):
        h = jnp.maximum(jnp.dot(h_ref[...], wp1_ref[...],
                                preferred_element_type=jnp.float32)
                        + bp1_ref[...], 0.0)
        logits = jnp.dot(h, wp2_ref[...],
                         preferred_element_type=jnp.float32) + bp2_ref[...]
        m = jnp.max(logits, axis=-1, keepdims=True)
        zl = logits - m
        lse = jnp.log(jnp.sum(jnp.exp(zl), axis=-1, keepdims=True))
        o_ref[...] = zl - lse

    return _tc_call(
        body, (N_PAD // BN,),
        [_rows(BN, H), _full((H, H)), _full((1, H)),
         _full((H, H)), _full((1, H))],
        _rows(BN, H),
        jax.ShapeDtypeStruct((N_PAD, H), jnp.float32))(hid, wp1, bp1, wp2, bp2)


def kernel(x, edge_index, W_proj, b_proj, W_m1, b_m1, W_m2, b_m2,
           W_mlp1, b_mlp1, W_mlp2, b_mlp2, W_o1, b_o1, W_o2, b_o2,
           ln_scale, ln_bias, W_red, b_red, W_p1, b_p1, W_p2, b_p2):
    f32 = jnp.float32
    src = edge_index[0]
    dst = edge_index[1]
    x_pad = jnp.zeros((N_PAD, H), f32).at[:N_NODES].set(x)

    perm = jnp.concatenate([jnp.arange(0, H, 2), jnp.arange(1, H, 2)])

    def cat_s(w, col_perm=False):
        w0, w1 = (w[0], w[1])
        if col_perm:
            w0, w1 = w0[:, perm], w1[:, perm]
        return jnp.concatenate([w0, w1], axis=1)

    def cat_b(b, col_perm=False):
        b0, b1 = (b[0], b[1])
        if col_perm:
            b0, b1 = b0[perm], b1[perm]
        return jnp.concatenate([b0, b1], axis=0)[None, :]

    nf = _tc_proj(x_pad, W_proj, b_proj[None, :])
    elist, ecnt = _sc_bin(dst)
    hidden = jnp.zeros((N_PAD, H), f32)
    for i in range(2):
        m1c, m2c, h1c = _tc_pre(
            nf, hidden,
            cat_s(W_m1[i], True), cat_b(b_m1[i], True),
            cat_s(W_m2[i], True), cat_b(b_m2[i], True),
            cat_s(W_o1[i]), cat_b(b_o1[i]))
        e = _sc_edge_gather(m1c, m2c, src, dst)
        p = _tc_edge_mlp(e, W_mlp1[i][:, perm, :], cat_b(b_mlp1[i]),
                         W_mlp2[i][:, :, perm], cat_b(b_mlp2[i], True))
        red = _sc_segmax(p, elist, ecnt)
        hidden = _tc_post(red, h1c, W_o2[i][:, perm, :], cat_b(b_o2[i]),
                          cat_b(ln_scale[i]), cat_b(ln_bias[i]),
                          W_red[i], b_red[i][None, :])
    out = _tc_head(hidden, W_p1, b_p1[None, :], _wp2pad(W_p2), _bp2pad(b_p2))
    return out[:N_NODES, :OUT_DIM]


def _wp2pad(w):
    return jnp.zeros((H, H), jnp.float32).at[:, :OUT_DIM].set(w)


def _bp2pad(b):
    return jnp.full((1, H), -1e30, jnp.float32).at[0, :OUT_DIM].set(b)


# speedup vs baseline: 1.0864x; 1.0864x over previous
import functools

import jax
import jax.numpy as jnp
from jax import lax
from jax.experimental import pallas as pl
from jax.experimental.pallas import tpu as pltpu
from jax.experimental.pallas import tpu_sc as plsc

N_NODES = 10000
N_PAD = 10240
E_EDGES = 320000
H = 128
HW = H // 2
OUT_DIM = 47

NW = 32
E_PER_W = E_EDGES // NW
GCHUNK = 200
NODES_PER_W = N_PAD // NW
DCHUNK = 2000

_sc_mesh = functools.partial(
    plsc.VectorSubcoreMesh, core_axis_name="c", subcore_axis_name="s")


def _wid():
    return lax.axis_index("s") * 2 + lax.axis_index("c")


NCH = E_PER_W // GCHUNK


@functools.partial(
    pl.kernel,
    mesh=_sc_mesh(),
    compiler_params=pltpu.CompilerParams(needs_layout_passes=False),
    out_type=jax.ShapeDtypeStruct((E_EDGES, 2 * HW), jnp.int32),
    scratch_types=[
        [pltpu.VMEM((GCHUNK,), jnp.int32)] * 2,
        [pltpu.VMEM((GCHUNK,), jnp.int32)] * 2,
        [pltpu.VMEM((GCHUNK, 2 * HW), jnp.int32)] * 2,
        [pltpu.VMEM((GCHUNK, 2 * HW), jnp.int32)] * 2,
        [pltpu.SemaphoreType.DMA] * 2,
        [pltpu.SemaphoreType.DMA] * 2,
    ],
)
def _sc_edge_gather(m1_hbm, m2_hbm, src_hbm, dst_hbm, out_hbm,
                    sidx, didx, b1, b2, gsem, osem):
    base = _wid() * E_PER_W

    def load_and_fire(j, par):
        off = pl.multiple_of(base + j * GCHUNK, 8)
        pltpu.sync_copy(src_hbm.at[pl.ds(off, GCHUNK)], sidx[par])
        pltpu.sync_copy(dst_hbm.at[pl.ds(off, GCHUNK)], didx[par])
        pltpu.async_copy(m1_hbm.at[sidx[par]], b1[par], gsem[par])
        pltpu.async_copy(m2_hbm.at[didx[par]], b2[par], gsem[par])

    load_and_fire(0, 0)

    def pair(k, carry):
        for par in range(2):
            j = 2 * k + par
            pltpu.make_async_copy(m1_hbm.at[sidx[par]], b1[par],
                                  gsem[par]).wait()
            pltpu.make_async_copy(m2_hbm.at[didx[par]], b2[par],
                                  gsem[par]).wait()

            nxt = 1 - par

            @pl.when(j + 1 < NCH)
            def _():
                @pl.when(j >= 1)
                def _():
                    pltpu.make_async_copy(
                        b1[nxt], out_hbm.at[pl.ds(0, GCHUNK)],
                        osem[nxt]).wait()

                load_and_fire(j + 1, nxt)

            def row(r, c2):
                for u in range(2 * HW // 16):
                    sl = pl.ds(u * 16, 16)
                    a = plsc.bitcast(b1[par][r, sl], jnp.bfloat16)
                    b = plsc.bitcast(b2[par][r, sl], jnp.bfloat16)
                    b1[par][r, sl] = plsc.bitcast(a + b, jnp.int32)
                return c2

            lax.fori_loop(0, GCHUNK, row, 0)
            off = pl.multiple_of(base + j * GCHUNK, 8)
            pltpu.async_copy(b1[par], out_hbm.at[pl.ds(off, GCHUNK)],
                             osem[par])
        return carry

    lax.fori_loop(0, NCH // 2, pair, 0)
    for par in range(2):
        pltpu.make_async_copy(b1[par], out_hbm.at[pl.ds(0, GCHUNK)],
                              osem[par]).wait()


RING = 4096
FBLK = 2048
EROUND = 158 * FBLK


@functools.partial(
    pl.kernel,
    mesh=_sc_mesh(),
    compiler_params=pltpu.CompilerParams(needs_layout_passes=False),
    out_type=(jax.ShapeDtypeStruct((NW * EROUND,), jnp.int32),
              jax.ShapeDtypeStruct((NW * 16,), jnp.int32)),
    scratch_types=[
        pltpu.VMEM((DCHUNK,), jnp.int32),
        pltpu.VMEM((RING,), jnp.int32),
        pltpu.VMEM((16,), jnp.int32),
    ],
)
def _sc_bin(dst_hbm, list_hbm, cnt_hbm, dbuf, ring, cbuf):
    wid = _wid()
    lo = wid * NODES_PER_W
    lane = lax.iota(jnp.int32, 16)
    lov = lax.broadcast_in_dim(lo, (16,), ())
    npw_vec = jnp.full((16,), NODES_PER_W, dtype=jnp.int32)
    dummy = jnp.full((16,), NODES_PER_W, dtype=jnp.int32)

    def init_ring(r, c):
        ring[pl.ds(r * 16, 16)] = dummy
        return c

    lax.fori_loop(0, RING // 16, init_ring, 0)

    def flush(state):
        cnt, flushed = state
        blk = flushed // FBLK

        def do(par):
            off = pl.multiple_of(wid * EROUND + flushed, FBLK)
            pltpu.sync_copy(ring.at[pl.ds(par * FBLK, FBLK)],
                            list_hbm.at[pl.ds(off, FBLK)])

        @pl.when(blk % 2 == 0)
        def _():
            do(0)

        @pl.when(blk % 2 == 1)
        def _():
            do(1)

        return cnt, flushed + FBLK

    def chunk(c, carry):
        cnt0, flushed, eid0 = carry
        pltpu.sync_copy(dst_hbm.at[pl.ds(c * DCHUNK, DCHUNK)], dbuf)

        def scan_group(g, st):
            cnt, eidv = st
            v = dbuf[pl.ds(g * 16, 16)]
            rel = v - lov
            m = jnp.bitwise_and(rel >= 0, rel < npw_vec)
            cs = jnp.cumsum(m.astype(jnp.int32))
            addr = jnp.bitwise_and(
                lax.broadcast_in_dim(cnt, (16,), ()) + cs - 1, RING - 1)
            packed = eidv * 512 + rel
            plsc.store_scatter(ring, [addr], packed, mask=m)
            return cnt + jnp.max(cs), eidv + 16

        cnt, eid0 = lax.fori_loop(0, DCHUNK // 16, scan_group, (cnt0, eid0))
        cnt, flushed = lax.cond(cnt - flushed >= FBLK, flush,
                                lambda s: s, (cnt, flushed))
        return cnt, flushed, eid0

    cnt, flushed, _ = lax.fori_loop(0, E_EDGES // DCHUNK, chunk,
                                    (jnp.int32(0), jnp.int32(0), lane))
    for _ in range(2):
        cnt, flushed = lax.cond(flushed < cnt, flush,
                                lambda s: s, (cnt, flushed))
    cbuf[...] = lax.broadcast_in_dim(cnt, (16,), ())
    pltpu.sync_copy(cbuf, cnt_hbm.at[pl.ds(pl.multiple_of(wid * 16, 16), 16)])


RB = 128
GPB = RB // 16


@functools.partial(
    pl.kernel,
    mesh=_sc_mesh(),
    compiler_params=pltpu.CompilerParams(needs_layout_passes=False),
    out_type=jax.ShapeDtypeStruct((N_PAD, 2 * HW), jnp.int32),
    scratch_types=[
        pltpu.VMEM((16,), jnp.int32),
        pltpu.VMEM((FBLK,), jnp.int32),
        pltpu.VMEM((FBLK,), jnp.int32),
        pltpu.VMEM((FBLK,), jnp.int32),
        pltpu.VMEM((RB, 2 * HW), jnp.int32),
        pltpu.VMEM((RB, 2 * HW), jnp.int32),
        pltpu.VMEM((NODES_PER_W + 1, 2 * HW), jnp.int32),
        pltpu.SemaphoreType.DMA,
        pltpu.SemaphoreType.DMA,
    ],
)
def _sc_segmax(p_hbm, list_hbm, cnt_hbm, red_hbm,
               cbuf, lbuf, idxb, relb, rb0, rb1, acc, sem0, sem1):
    wid = _wid()
    lo = wid * NODES_PER_W
    neg_inf = jnp.full((16,), 0xFF80FF80 - (1 << 32), dtype=jnp.int32)

    def init_row(r, c):
        for u in range(2 * HW // 16):
            acc[r, pl.ds(u * 16, 16)] = neg_inf
        return c

    lax.fori_loop(0, NODES_PER_W + 1, init_row, 0)

    pltpu.sync_copy(cnt_hbm.at[pl.ds(pl.multiple_of(wid * 16, 16), 16)], cbuf)
    cnt = cbuf[pl.ds(0, 16)][0]
    nlb = (cnt + FBLK - 1) // FBLK

    rbufs = (rb0, rb1)
    sems = (sem0, sem1)

    def fire(b, par):
        pltpu.async_copy(p_hbm.at[idxb.at[pl.ds(b * RB, RB)]],
                         rbufs[par], sems[par])

    def wait(par):
        pltpu.make_async_copy(p_hbm.at[idxb.at[pl.ds(0, RB)]],
                              rbufs[par], sems[par]).wait()

    def lblock(bidx, c0):
        loff = pl.multiple_of(wid * EROUND + bidx * FBLK, FBLK)
        pltpu.sync_copy(list_hbm.at[pl.ds(loff, FBLK)], lbuf)

        def decode(g, c):
            u = lbuf[pl.ds(g * 16, 16)]
            idxb[pl.ds(g * 16, 16)] = lax.shift_right_logical(u, 9)
            relb[pl.ds(g * 16, 16)] = jnp.bitwise_and(u, 511)
            return c

        lax.fori_loop(0, FBLK // 16, decode, 0)
        rem = jnp.minimum(cnt - bidx * FBLK, FBLK)
        ngroups = (rem + 15) // 16
        nb = (rem + RB - 1) // RB

        @pl.when(nb > 0)
        def _():
            fire(0, 0)

        @pl.when(nb > 1)
        def _():
            fire(1, 1)

        def bpair(k, c):
            for par in range(2):
                b = 2 * k + par

                @pl.when(b < nb)
                def _():
                    wait(par)
                    gend = jnp.minimum((b + 1) * GPB, ngroups)

                    def group(g, c2):
                        gl = g - b * GPB
                        relv = relb[pl.ds(g * 16, 16)]
                        for j in range(16):
                            rel = relv[j]
                            row = gl * 16 + j
                            for u in range(2 * HW // 16):
                                sl = pl.ds(u * 16, 16)
                                a = plsc.bitcast(acc[rel, sl], jnp.bfloat16)
                                r = plsc.bitcast(rbufs[par][row, sl],
                                                 jnp.bfloat16)
                                acc[rel, sl] = plsc.bitcast(
                                    jnp.maximum(a, r), jnp.int32)
                        return c2

                    lax.fori_loop(b * GPB, gend, group, 0)

                    @pl.when(b + 2 < nb)
                    def _():
                        fire(b + 2, par)
            return c

        lax.fori_loop(0, (nb + 1) // 2, bpair, 0)
        return c0

    lax.fori_loop(0, nlb, lblock, 0)
    pltpu.sync_copy(acc.at[pl.ds(0, NODES_PER_W)],
                    red_hbm.at[pl.ds(lo, NODES_PER_W)])


BN = 256
BE = 512


def _unpack_half(words_i32):
    u = lax.bitcast_convert_type(words_i32, jnp.uint32)
    even = lax.bitcast_convert_type(u << 16, jnp.float32)
    odd = lax.bitcast_convert_type(u & jnp.uint32(0xFFFF0000), jnp.float32)
    return jnp.concatenate([even, odd], axis=1)


def _pack_half(vals_f32):
    def rne(x):
        u = lax.bitcast_convert_type(x, jnp.uint32)
        return (u + jnp.uint32(0x7FFF) + ((u >> 16) & jnp.uint32(1))) >> 16

    ev = rne(vals_f32[:, :HW])
    od = rne(vals_f32[:, HW:])
    return lax.bitcast_convert_type(ev | (od << 16), jnp.int32)


def _tc_call(body, grid, in_specs, out_specs, out_shape):
    return pl.pallas_call(body, grid=grid, in_specs=in_specs,
                          out_specs=out_specs, out_shape=out_shape)


def _full(shape):
    return pl.BlockSpec(shape, lambda i: (0,) * len(shape))


def _rows(block, width):
    return pl.BlockSpec((block, width), lambda i: (i, 0))


def _tc_proj(x, w, b):
    def body(x_ref, w_ref, b_ref, o_ref):
        o_ref[...] = jnp.dot(x_ref[...], w_ref[...],
                             preferred_element_type=jnp.float32) + b_ref[...]
    return _tc_call(
        body, (N_PAD // BN,),
        [_rows(BN, H), _full((H, H)), _full((1, H))],
        _rows(BN, H), jax.ShapeDtypeStruct((N_PAD, H), jnp.float32))(x, w, b)


def _tc_pre(nf, hid, wm1, bm1, wm2, bm2, wo1, bo1):
    def body(nf_ref, hid_ref, wm1_ref, bm1_ref, wm2_ref, bm2_ref,
             wo1_ref, bo1_ref, m1_ref, m2_ref, h1_ref):
        z = jnp.concatenate([nf_ref[...], hid_ref[...]], axis=1)
        m1 = jnp.dot(z, wm1_ref[...],
                     preferred_element_type=jnp.float32) + bm1_ref[...]
        m2 = jnp.dot(z, wm2_ref[...],
                     preferred_element_type=jnp.float32) + bm2_ref[...]
        for s in range(2):
            sl = slice(s * H, (s + 1) * H)
            m1_ref[:, s * HW:(s + 1) * HW] = _pack_half(m1[:, sl])
            m2_ref[:, s * HW:(s + 1) * HW] = _pack_half(m2[:, sl])
        h1_ref[...] = jnp.dot(z, wo1_ref[...],
                              preferred_element_type=jnp.float32) + bo1_ref[...]

    shp_pk = jax.ShapeDtypeStruct((N_PAD, 2 * HW), jnp.int32)
    shp32 = jax.ShapeDtypeStruct((N_PAD, 2 * H), jnp.float32)
    return _tc_call(
        body, (N_PAD // BN,),
        [_rows(BN, H), _rows(BN, H),
         _full((2 * H, 2 * H)), _full((1, 2 * H)),
         _full((2 * H, 2 * H)), _full((1, 2 * H)),
         _full((2 * H, 2 * H)), _full((1, 2 * H))],
        (_rows(BN, 2 * HW), _rows(BN, 2 * HW), _rows(BN, 2 * H)),
        (shp_pk, shp_pk, shp32),
    )(nf, hid, wm1, bm1, wm2, bm2, wo1, bo1)


def _tc_edge_mlp(e, w1, b1, w2, b2):
    def body(e_ref, w1_ref, b1_ref, w2_ref, b2_ref, o_ref):
        for s in range(2):
            sl = slice(s * H, (s + 1) * H)
            wsl = slice(s * HW, (s + 1) * HW)
            msgs = jnp.maximum(_unpack_half(e_ref[:, wsl]),
                               0.0).astype(jnp.bfloat16)
            t = jnp.maximum(
                jnp.dot(msgs, w1_ref[s], preferred_element_type=jnp.float32)
                + b1_ref[:, sl], 0.0).astype(jnp.bfloat16)
            out = jnp.dot(t, w2_ref[s],
                          preferred_element_type=jnp.float32) + b2_ref[:, sl]
            o_ref[:, wsl] = _pack_half(out)

    return _tc_call(
        body, (E_EDGES // BE,),
        [_rows(BE, 2 * HW), _full((2, H, H)), _full((1, 2 * H)),
         _full((2, H, H)), _full((1, 2 * H))],
        _rows(BE, 2 * HW),
        jax.ShapeDtypeStruct((E_EDGES, 2 * HW), jnp.int32))(e, w1, b1, w2, b2)


def _tc_post(red, h1cat, wo2, bo2, ln_s, ln_b, wred, bred):
    def body(red_ref, h1_ref, wo2_ref, bo2_ref, lns_ref, lnb_ref,
             wred_ref, bred_ref, o_ref):
        outs = []
        for s in range(2):
            sl = slice(s * H, (s + 1) * H)
            r = _unpack_half(red_ref[:, s * HW:(s + 1) * HW])
            r = jnp.where(jnp.isfinite(r), r, 0.0)
            h2 = jnp.dot(r, wo2_ref[s],
                         preferred_element_type=jnp.float32) + bo2_ref[:, sl]
            ret = jnp.maximum(h1_ref[:, sl] + h2, 0.0)
            mu = jnp.mean(ret, axis=-1, keepdims=True)
            d = ret - mu
            var = jnp.mean(d * d, axis=-1, keepdims=True)
            ret = d / jnp.sqrt(var + 1e-5) * lns_ref[:, sl] + lnb_ref[:, sl]
            outs.append(ret)
        cat = jnp.concatenate(outs, axis=1)
        o_ref[...] = jnp.dot(cat, wred_ref[...],
                             preferred_element_type=jnp.float32) + bred_ref[...]

    return _tc_call(
        body, (N_PAD // BN,),
        [_rows(BN, 2 * HW), _rows(BN, 2 * H), _full((2, H, H)),
         _full((1, 2 * H)), _full((1, 2 * H)), _full((1, 2 * H)),
         _full((2 * H, H)), _full((1, H))],
        _rows(BN, H),
        jax.ShapeDtypeStruct((N_PAD, H), jnp.float32),
    )(red, h1cat, wo2, bo2, ln_s, ln_b, wred, bred)


def _tc_head(hid, wp1, bp1, wp2, bp2):
    def body(h_ref, wp1_ref, bp1_ref, wp2_ref, bp2_ref, o_ref):
        h = jnp.maximum(jnp.dot(h_ref[...], wp1_ref[...],
                                preferred_element_type=jnp.float32)
                        + bp1_ref[...], 0.0)
        logits = jnp.dot(h, wp2_ref[...],
                         preferred_element_type=jnp.float32) + bp2_ref[...]
        m = jnp.max(logits, axis=-1, keepdims=True)
        zl = logits - m
        lse = jnp.log(jnp.sum(jnp.exp(zl), axis=-1, keepdims=True))
        o_ref[...] = zl - lse

    return _tc_call(
        body, (N_PAD // BN,),
        [_rows(BN, H), _full((H, H)), _full((1, H)),
         _full((H, H)), _full((1, H))],
        _rows(BN, H),
        jax.ShapeDtypeStruct((N_PAD, H), jnp.float32))(hid, wp1, bp1, wp2, bp2)


def kernel(x, edge_index, W_proj, b_proj, W_m1, b_m1, W_m2, b_m2,
           W_mlp1, b_mlp1, W_mlp2, b_mlp2, W_o1, b_o1, W_o2, b_o2,
           ln_scale, ln_bias, W_red, b_red, W_p1, b_p1, W_p2, b_p2):
    f32 = jnp.float32
    src = edge_index[0]
    dst = edge_index[1]
    x_pad = jnp.zeros((N_PAD, H), f32).at[:N_NODES].set(x)

    perm = jnp.concatenate([jnp.arange(0, H, 2), jnp.arange(1, H, 2)])

    def cat_s(w, col_perm=False):
        w0, w1 = (w[0], w[1])
        if col_perm:
            w0, w1 = w0[:, perm], w1[:, perm]
        return jnp.concatenate([w0, w1], axis=1)

    def cat_b(b, col_perm=False):
        b0, b1 = (b[0], b[1])
        if col_perm:
            b0, b1 = b0[perm], b1[perm]
        return jnp.concatenate([b0, b1], axis=0)[None, :]

    nf = _tc_proj(x_pad, W_proj, b_proj[None, :])
    elist, ecnt = _sc_bin(dst)
    hidden = jnp.zeros((N_PAD, H), f32)
    for i in range(2):
        m1c, m2c, h1c = _tc_pre(
            nf, hidden,
            cat_s(W_m1[i], True), cat_b(b_m1[i], True),
            cat_s(W_m2[i], True), cat_b(b_m2[i], True),
            cat_s(W_o1[i]), cat_b(b_o1[i]))
        e = _sc_edge_gather(m1c, m2c, src, dst)
        p = _tc_edge_mlp(e, W_mlp1[i][:, perm, :].astype(jnp.bfloat16),
                         cat_b(b_mlp1[i]),
                         W_mlp2[i][:, :, perm].astype(jnp.bfloat16),
                         cat_b(b_mlp2[i], True))
        red = _sc_segmax(p, elist, ecnt)
        hidden = _tc_post(red, h1c, W_o2[i][:, perm, :], cat_b(b_o2[i]),
                          cat_b(ln_scale[i]), cat_b(ln_bias[i]),
                          W_red[i], b_red[i][None, :])
    out = _tc_head(hidden, W_p1, b_p1[None, :], _wp2pad(W_p2), _bp2pad(b_p2))
    return out[:N_NODES, :OUT_DIM]


def _wp2pad(w):
    return jnp.zeros((H, H), jnp.float32).at[:, :OUT_DIM].set(w)


def _bp2pad(b):
    return jnp.full((1, H), -1e30, jnp.float32).at[0, :OUT_DIM].set(b)


# speedup vs baseline: 1.0877x; 1.0011x over previous
import functools

import jax
import jax.numpy as jnp
from jax import lax
from jax.experimental import pallas as pl
from jax.experimental.pallas import tpu as pltpu
from jax.experimental.pallas import tpu_sc as plsc

N_NODES = 10000
N_PAD = 10240
E_EDGES = 320000
H = 128
HW = H // 2
OUT_DIM = 47

NW = 32
E_PER_W = E_EDGES // NW
GCHUNK = 200
NODES_PER_W = N_PAD // NW
DCHUNK = 2000

_sc_mesh = functools.partial(
    plsc.VectorSubcoreMesh, core_axis_name="c", subcore_axis_name="s")


def _wid():
    return lax.axis_index("s") * 2 + lax.axis_index("c")


NCH = E_PER_W // GCHUNK


@functools.partial(
    pl.kernel,
    mesh=_sc_mesh(),
    compiler_params=pltpu.CompilerParams(needs_layout_passes=False),
    out_type=jax.ShapeDtypeStruct((E_EDGES, 2 * HW), jnp.int32),
    scratch_types=[
        [pltpu.VMEM((GCHUNK,), jnp.int32)] * 2,
        [pltpu.VMEM((GCHUNK,), jnp.int32)] * 2,
        [pltpu.VMEM((GCHUNK, 2 * HW), jnp.int32)] * 2,
        [pltpu.VMEM((GCHUNK, 2 * HW), jnp.int32)] * 2,
        [pltpu.SemaphoreType.DMA] * 2,
        [pltpu.SemaphoreType.DMA] * 2,
    ],
)
def _sc_edge_gather(m1_hbm, m2_hbm, src_hbm, dst_hbm, out_hbm,
                    sidx, didx, b1, b2, gsem, osem):
    base = _wid() * E_PER_W

    def load_and_fire(j, par):
        off = pl.multiple_of(base + j * GCHUNK, 8)
        pltpu.sync_copy(src_hbm.at[pl.ds(off, GCHUNK)], sidx[par])
        pltpu.sync_copy(dst_hbm.at[pl.ds(off, GCHUNK)], didx[par])
        pltpu.async_copy(m1_hbm.at[sidx[par]], b1[par], gsem[par])
        pltpu.async_copy(m2_hbm.at[didx[par]], b2[par], gsem[par])

    load_and_fire(0, 0)

    def pair(k, carry):
        for par in range(2):
            j = 2 * k + par
            pltpu.make_async_copy(m1_hbm.at[sidx[par]], b1[par],
                                  gsem[par]).wait()
            pltpu.make_async_copy(m2_hbm.at[didx[par]], b2[par],
                                  gsem[par]).wait()

            nxt = 1 - par

            @pl.when(j + 1 < NCH)
            def _():
                @pl.when(j >= 1)
                def _():
                    pltpu.make_async_copy(
                        b1[nxt], out_hbm.at[pl.ds(0, GCHUNK)],
                        osem[nxt]).wait()

                load_and_fire(j + 1, nxt)

            def row(r, c2):
                for u in range(2 * HW // 16):
                    sl = pl.ds(u * 16, 16)
                    a = plsc.bitcast(b1[par][r, sl], jnp.bfloat16)
                    b = plsc.bitcast(b2[par][r, sl], jnp.bfloat16)
                    b1[par][r, sl] = plsc.bitcast(a + b, jnp.int32)
                return c2

            lax.fori_loop(0, GCHUNK, row, 0)
            off = pl.multiple_of(base + j * GCHUNK, 8)
            pltpu.async_copy(b1[par], out_hbm.at[pl.ds(off, GCHUNK)],
                             osem[par])
        return carry

    lax.fori_loop(0, NCH // 2, pair, 0)
    for par in range(2):
        pltpu.make_async_copy(b1[par], out_hbm.at[pl.ds(0, GCHUNK)],
                              osem[par]).wait()


RING = 4096
FBLK = 2048
EROUND = 158 * FBLK


@functools.partial(
    pl.kernel,
    mesh=_sc_mesh(),
    compiler_params=pltpu.CompilerParams(needs_layout_passes=False),
    out_type=(jax.ShapeDtypeStruct((NW * EROUND,), jnp.int32),
              jax.ShapeDtypeStruct((NW * 16,), jnp.int32)),
    scratch_types=[
        pltpu.VMEM((DCHUNK,), jnp.int32),
        pltpu.VMEM((RING,), jnp.int32),
        pltpu.VMEM((16,), jnp.int32),
    ],
)
def _sc_bin(dst_hbm, list_hbm, cnt_hbm, dbuf, ring, cbuf):
    wid = _wid()
    lo = wid * NODES_PER_W
    lane = lax.iota(jnp.int32, 16)
    lov = lax.broadcast_in_dim(lo, (16,), ())
    npw_vec = jnp.full((16,), NODES_PER_W, dtype=jnp.int32)
    dummy = jnp.full((16,), NODES_PER_W, dtype=jnp.int32)

    def init_ring(r, c):
        ring[pl.ds(r * 16, 16)] = dummy
        return c

    lax.fori_loop(0, RING // 16, init_ring, 0)

    def flush(state):
        cnt, flushed = state
        blk = flushed // FBLK

        def do(par):
            off = pl.multiple_of(wid * EROUND + flushed, FBLK)
            pltpu.sync_copy(ring.at[pl.ds(par * FBLK, FBLK)],
                            list_hbm.at[pl.ds(off, FBLK)])

        @pl.when(blk % 2 == 0)
        def _():
            do(0)

        @pl.when(blk % 2 == 1)
        def _():
            do(1)

        return cnt, flushed + FBLK

    def chunk(c, carry):
        cnt0, flushed, eid0 = carry
        pltpu.sync_copy(dst_hbm.at[pl.ds(c * DCHUNK, DCHUNK)], dbuf)

        def scan_group(g, st):
            cnt, eidv = st
            v = dbuf[pl.ds(g * 16, 16)]
            rel = v - lov
            m = jnp.bitwise_and(rel >= 0, rel < npw_vec)
            cs = jnp.cumsum(m.astype(jnp.int32))
            addr = jnp.bitwise_and(
                lax.broadcast_in_dim(cnt, (16,), ()) + cs - 1, RING - 1)
            packed = eidv * 512 + rel
            plsc.store_scatter(ring, [addr], packed, mask=m)
            return cnt + jnp.max(cs), eidv + 16

        cnt, eid0 = lax.fori_loop(0, DCHUNK // 16, scan_group, (cnt0, eid0))
        cnt, flushed = lax.cond(cnt - flushed >= FBLK, flush,
                                lambda s: s, (cnt, flushed))
        return cnt, flushed, eid0

    cnt, flushed, _ = lax.fori_loop(0, E_EDGES // DCHUNK, chunk,
                                    (jnp.int32(0), jnp.int32(0), lane))
    for _ in range(2):
        cnt, flushed = lax.cond(flushed < cnt, flush,
                                lambda s: s, (cnt, flushed))
    cbuf[...] = lax.broadcast_in_dim(cnt, (16,), ())
    pltpu.sync_copy(cbuf, cnt_hbm.at[pl.ds(pl.multiple_of(wid * 16, 16), 16)])


RB = 128
GPB = RB // 16


@functools.partial(
    pl.kernel,
    mesh=_sc_mesh(),
    compiler_params=pltpu.CompilerParams(needs_layout_passes=False),
    out_type=jax.ShapeDtypeStruct((N_PAD, 2 * HW), jnp.int32),
    scratch_types=[
        pltpu.VMEM((16,), jnp.int32),
        pltpu.VMEM((FBLK,), jnp.int32),
        pltpu.VMEM((FBLK,), jnp.int32),
        pltpu.VMEM((FBLK,), jnp.int32),
        pltpu.VMEM((RB, 2 * HW), jnp.int32),
        pltpu.VMEM((RB, 2 * HW), jnp.int32),
        pltpu.VMEM((NODES_PER_W + 1, 2 * HW), jnp.int32),
        pltpu.SemaphoreType.DMA,
        pltpu.SemaphoreType.DMA,
    ],
)
def _sc_segmax(p_hbm, list_hbm, cnt_hbm, red_hbm,
               cbuf, lbuf, idxb, relb, rb0, rb1, acc, sem0, sem1):
    wid = _wid()
    lo = wid * NODES_PER_W
    neg_inf = jnp.full((16,), 0xFF80FF80 - (1 << 32), dtype=jnp.int32)

    def init_row(r, c):
        for u in range(2 * HW // 16):
            acc[r, pl.ds(u * 16, 16)] = neg_inf
        return c

    lax.fori_loop(0, NODES_PER_W + 1, init_row, 0)

    pltpu.sync_copy(cnt_hbm.at[pl.ds(pl.multiple_of(wid * 16, 16), 16)], cbuf)
    cnt = cbuf[pl.ds(0, 16)][0]
    nlb = (cnt + FBLK - 1) // FBLK

    rbufs = (rb0, rb1)
    sems = (sem0, sem1)

    def fire(b, par):
        pltpu.async_copy(p_hbm.at[idxb.at[pl.ds(b * RB, RB)]],
                         rbufs[par], sems[par])

    def wait(par):
        pltpu.make_async_copy(p_hbm.at[idxb.at[pl.ds(0, RB)]],
                              rbufs[par], sems[par]).wait()

    def lblock(bidx, c0):
        loff = pl.multiple_of(wid * EROUND + bidx * FBLK, FBLK)
        pltpu.sync_copy(list_hbm.at[pl.ds(loff, FBLK)], lbuf)

        def decode(g, c):
            u = lbuf[pl.ds(g * 16, 16)]
            idxb[pl.ds(g * 16, 16)] = lax.shift_right_logical(u, 9)
            relb[pl.ds(g * 16, 16)] = jnp.bitwise_and(u, 511)
            return c

        lax.fori_loop(0, FBLK // 16, decode, 0)
        rem = jnp.minimum(cnt - bidx * FBLK, FBLK)
        ngroups = (rem + 15) // 16
        nb = (rem + RB - 1) // RB

        @pl.when(nb > 0)
        def _():
            fire(0, 0)

        @pl.when(nb > 1)
        def _():
            fire(1, 1)

        def bpair(k, c):
            for par in range(2):
                b = 2 * k + par

                @pl.when(b < nb)
                def _():
                    wait(par)
                    gend = jnp.minimum((b + 1) * GPB, ngroups)

                    def group(g, c2):
                        gl = g - b * GPB
                        relv = relb[pl.ds(g * 16, 16)]
                        for j in range(16):
                            rel = relv[j]
                            row = gl * 16 + j
                            for u in range(2 * HW // 16):
                                sl = pl.ds(u * 16, 16)
                                a = plsc.bitcast(acc[rel, sl], jnp.bfloat16)
                                r = plsc.bitcast(rbufs[par][row, sl],
                                                 jnp.bfloat16)
                                acc[rel, sl] = plsc.bitcast(
                                    jnp.maximum(a, r), jnp.int32)
                        return c2

                    lax.fori_loop(b * GPB, gend, group, 0)

                    @pl.when(b + 2 < nb)
                    def _():
                        fire(b + 2, par)
            return c

        lax.fori_loop(0, (nb + 1) // 2, bpair, 0)
        return c0

    lax.fori_loop(0, nlb, lblock, 0)
    pltpu.sync_copy(acc.at[pl.ds(0, NODES_PER_W)],
                    red_hbm.at[pl.ds(lo, NODES_PER_W)])


BN = 256
BE = 512


def _unpack_half(words_i32):
    u = lax.bitcast_convert_type(words_i32, jnp.uint32)
    even = lax.bitcast_convert_type(u << 16, jnp.float32)
    odd = lax.bitcast_convert_type(u & jnp.uint32(0xFFFF0000), jnp.float32)
    return jnp.concatenate([even, odd], axis=1)


def _pack_half(vals_f32):
    def rne(x):
        u = lax.bitcast_convert_type(x, jnp.uint32)
        return (u + jnp.uint32(0x7FFF) + ((u >> 16) & jnp.uint32(1))) >> 16

    ev = rne(vals_f32[:, :HW])
    od = rne(vals_f32[:, HW:])
    return lax.bitcast_convert_type(ev | (od << 16), jnp.int32)


def _tc_call(body, grid, in_specs, out_specs, out_shape):
    return pl.pallas_call(body, grid=grid, in_specs=in_specs,
                          out_specs=out_specs, out_shape=out_shape)


def _full(shape):
    return pl.BlockSpec(shape, lambda i: (0,) * len(shape))


def _rows(block, width):
    return pl.BlockSpec((block, width), lambda i: (i, 0))


def _tc_proj(x, w, b):
    def body(x_ref, w_ref, b_ref, o_ref):
        o_ref[...] = jnp.dot(x_ref[...], w_ref[...],
                             preferred_element_type=jnp.float32) + b_ref[...]
    return _tc_call(
        body, (N_PAD // BN,),
        [_rows(BN, H), _full((H, H)), _full((1, H))],
        _rows(BN, H), jax.ShapeDtypeStruct((N_PAD, H), jnp.float32))(x, w, b)


def _tc_pre(nf, hid, wm1, bm1, wm2, bm2, wo1, bo1):
    def body(nf_ref, hid_ref, wm1_ref, bm1_ref, wm2_ref, bm2_ref,
             wo1_ref, bo1_ref, m1_ref, m2_ref, h1_ref):
        z = jnp.concatenate([nf_ref[...], hid_ref[...]], axis=1)
        m1 = jnp.dot(z, wm1_ref[...],
                     preferred_element_type=jnp.float32) + bm1_ref[...]
        m2 = jnp.dot(z, wm2_ref[...],
                     preferred_element_type=jnp.float32) + bm2_ref[...]
        for s in range(2):
            sl = slice(s * H, (s + 1) * H)
            m1_ref[:, s * HW:(s + 1) * HW] = _pack_half(m1[:, sl])
            m2_ref[:, s * HW:(s + 1) * HW] = _pack_half(m2[:, sl])
        h1_ref[...] = jnp.dot(z, wo1_ref[...],
                              preferred_element_type=jnp.float32) + bo1_ref[...]

    shp_pk = jax.ShapeDtypeStruct((N_PAD, 2 * HW), jnp.int32)
    shp32 = jax.ShapeDtypeStruct((N_PAD, 2 * H), jnp.float32)
    return _tc_call(
        body, (N_PAD // BN,),
        [_rows(BN, H), _rows(BN, H),
         _full((2 * H, 2 * H)), _full((1, 2 * H)),
         _full((2 * H, 2 * H)), _full((1, 2 * H)),
         _full((2 * H, 2 * H)), _full((1, 2 * H))],
        (_rows(BN, 2 * HW), _rows(BN, 2 * HW), _rows(BN, 2 * H)),
        (shp_pk, shp_pk, shp32),
    )(nf, hid, wm1, bm1, wm2, bm2, wo1, bo1)


def _tc_edge_mlp(e, w1, b1, w2, b2):
    def body(e_ref, w1_ref, b1_ref, w2_ref, b2_ref, o_ref):
        for s in range(2):
            sl = slice(s * H, (s + 1) * H)
            wsl = slice(s * HW, (s + 1) * HW)
            msgs = jnp.maximum(_unpack_half(e_ref[:, wsl]),
                               0.0).astype(jnp.bfloat16)
            t = jnp.maximum(
                jnp.dot(msgs, w1_ref[s], preferred_element_type=jnp.float32)
                + b1_ref[:, sl], 0.0).astype(jnp.bfloat16)
            out = jnp.dot(t, w2_ref[s],
                          preferred_element_type=jnp.float32) + b2_ref[:, sl]
            o_ref[:, wsl] = _pack_half(out)

    return _tc_call(
        body, (E_EDGES // BE,),
        [_rows(BE, 2 * HW), _full((2, H, H)), _full((1, 2 * H)),
         _full((2, H, H)), _full((1, 2 * H))],
        _rows(BE, 2 * HW),
        jax.ShapeDtypeStruct((E_EDGES, 2 * HW), jnp.int32))(e, w1, b1, w2, b2)


def _tc_post(red, h1cat, wo2, bo2, ln_s, ln_b, wred, bred):
    def body(red_ref, h1_ref, wo2_ref, bo2_ref, lns_ref, lnb_ref,
             wred_ref, bred_ref, o_ref):
        outs = []
        for s in range(2):
            sl = slice(s * H, (s + 1) * H)
            r = _unpack_half(red_ref[:, s * HW:(s + 1) * HW])
            r = jnp.where(jnp.isfinite(r), r, 0.0)
            h2 = jnp.dot(r, wo2_ref[s],
                         preferred_element_type=jnp.float32) + bo2_ref[:, sl]
            ret = jnp.maximum(h1_ref[:, sl] + h2, 0.0)
            mu = jnp.mean(ret, axis=-1, keepdims=True)
            d = ret - mu
            var = jnp.mean(d * d, axis=-1, keepdims=True)
            ret = d / jnp.sqrt(var + 1e-5) * lns_ref[:, sl] + lnb_ref[:, sl]
            outs.append(ret)
        cat = jnp.concatenate(outs, axis=1)
        o_ref[...] = jnp.dot(cat, wred_ref[...],
                             preferred_element_type=jnp.float32) + bred_ref[...]

    return _tc_call(
        body, (N_PAD // BN,),
        [_rows(BN, 2 * HW), _rows(BN, 2 * H), _full((2, H, H)),
         _full((1, 2 * H)), _full((1, 2 * H)), _full((1, 2 * H)),
         _full((2 * H, H)), _full((1, H))],
        _rows(BN, H),
        jax.ShapeDtypeStruct((N_PAD, H), jnp.float32),
    )(red, h1cat, wo2, bo2, ln_s, ln_b, wred, bred)


def _tc_head(hid, wp1, bp1, wp2, bp2):
    def body(h_ref, wp1_ref, bp1_ref, wp2_ref, bp2_ref, o_ref):
        h = jnp.maximum(jnp.dot(h_ref[...], wp1_ref[...],
                                preferred_element_type=jnp.float32)
                        + bp1_ref[...], 0.0)
        logits = jnp.dot(h, wp2_ref[...],
                         preferred_element_type=jnp.float32) + bp2_ref[...]
        m = jnp.max(logits, axis=-1, keepdims=True)
        zl = logits - m
        lse = jnp.log(jnp.sum(jnp.exp(zl), axis=-1, keepdims=True))
        o_ref[...] = zl - lse

    return _tc_call(
        body, (N_PAD // BN,),
        [_rows(BN, H), _full((H, H)), _full((1, H)),
         _full((H, H)), _full((1, H))],
        _rows(BN, H),
        jax.ShapeDtypeStruct((N_PAD, H), jnp.float32))(hid, wp1, bp1, wp2, bp2)


def kernel(x, edge_index, W_proj, b_proj, W_m1, b_m1, W_m2, b_m2,
           W_mlp1, b_mlp1, W_mlp2, b_mlp2, W_o1, b_o1, W_o2, b_o2,
           ln_scale, ln_bias, W_red, b_red, W_p1, b_p1, W_p2, b_p2):
    f32 = jnp.float32
    src = edge_index[0]
    dst = edge_index[1]
    x_pad = jnp.zeros((N_PAD, H), f32).at[:N_NODES].set(x)

    perm = jnp.concatenate([jnp.arange(0, H, 2), jnp.arange(1, H, 2)])

    def cat_s(w, col_perm=False):
        w0, w1 = (w[0], w[1])
        if col_perm:
            w0, w1 = w0[:, perm], w1[:, perm]
        return jnp.concatenate([w0, w1], axis=1)

    def cat_b(b, col_perm=False):
        b0, b1 = (b[0], b[1])
        if col_perm:
            b0, b1 = b0[perm], b1[perm]
        return jnp.concatenate([b0, b1], axis=0)[None, :]

    nf = _tc_proj(x_pad, W_proj, b_proj[None, :])
    elist = ecnt = None
    hidden = jnp.zeros((N_PAD, H), f32)
    for i in range(2):
        m1c, m2c, h1c = _tc_pre(
            nf, hidden,
            cat_s(W_m1[i], True), cat_b(b_m1[i], True),
            cat_s(W_m2[i], True), cat_b(b_m2[i], True),
            cat_s(W_o1[i]), cat_b(b_o1[i]))
        e = _sc_edge_gather(m1c, m2c, src, dst)
        if elist is None:
            elist, ecnt = _sc_bin(dst)
        p = _tc_edge_mlp(e, W_mlp1[i][:, perm, :].astype(jnp.bfloat16),
                         cat_b(b_mlp1[i]),
                         W_mlp2[i][:, :, perm].astype(jnp.bfloat16),
                         cat_b(b_mlp2[i], True))
        red = _sc_segmax(p, elist, ecnt)
        hidden = _tc_post(red, h1c, W_o2[i][:, perm, :], cat_b(b_o2[i]),
                          cat_b(ln_scale[i]), cat_b(ln_bias[i]),
                          W_red[i], b_red[i][None, :])
    out = _tc_head(hidden, W_p1, b_p1[None, :], _wp2pad(W_p2), _bp2pad(b_p2))
    return out[:N_NODES, :OUT_DIM]


def _wp2pad(w):
    return jnp.zeros((H, H), jnp.float32).at[:, :OUT_DIM].set(w)


def _bp2pad(b):
    return jnp.full((1, H), -1e30, jnp.float32).at[0, :OUT_DIM].set(b)
